# baseline (speedup 1.0000x reference)
import functools

import jax
import jax.numpy as jnp
from jax import lax
from jax.experimental import pallas as pl
from jax.experimental.pallas import tpu as pltpu
from jax.experimental.pallas import tpu_sc as plsc

_NC = 2
_NS = 16
_NW = _NC * _NS
_T = 100
_TP = 104
_D = 128
_K = 4
_CHUNK = 128
_NGC = _CHUNK // _K


def _sc_body(idx_hbm, tab_hbm, out_hbm, idx_v, rows_v, gs_a, gs_b, ws_a, ws_b):
    n_slabs = out_hbm.shape[0]
    per_w = n_slabs // _NW
    w = lax.axis_index("s") * _NC + lax.axis_index("c")
    base = w * per_w
    n_chunks = per_w // _CHUNK

    def gsem(g):
        return gs_a if g % 2 == 0 else gs_b

    def wsem(g):
        return ws_a if g % 2 == 0 else ws_b

    def buf(g, k):
        return ((g % 2) * _K + k) * _TP

    def fire_gathers(g_static, g_dyn):
        for k in range(_K):
            iview = idx_v.at[pl.ds((g_dyn * _K + k) * _TP, _TP)]
            pltpu.async_copy(tab_hbm.at[iview], rows_v.at[pl.ds(buf(g_static, k), _TP)],
                             gsem(g_static))

    def drain_gathers(g_static):
        for k in range(_K):
            pltpu.make_async_copy(tab_hbm.at[idx_v.at[pl.ds(0, _TP)]],
                                  rows_v.at[pl.ds(buf(g_static, k), _TP)],
                                  gsem(g_static)).wait()

    def fire_writes(c, g_static, g_dyn):
        for k in range(_K):
            slab = base + c * _CHUNK + g_dyn * _K + k
            pltpu.async_copy(rows_v.at[pl.ds(buf(g_static, k), _T)],
                             out_hbm.at[slab], wsem(g_static))

    def drain_writes(g_static):
        for k in range(_K):
            pltpu.make_async_copy(rows_v.at[pl.ds(buf(g_static, k), _T)],
                                  out_hbm.at[base], wsem(g_static)).wait()

    for c in range(n_chunks):
        pltpu.sync_copy(idx_hbm.at[pl.ds((base + c * _CHUNK) * _TP, _CHUNK * _TP)],
                        idx_v)
        fire_gathers(0, 0)
        fire_gathers(1, 1)

        def pair(m, carry):
            g0 = 2 * m
            g1 = g0 + 1
            drain_gathers(0)
            fire_writes(c, 0, g0)
            drain_gathers(1)
            drain_writes(0)
            fire_gathers(0, g0 + 2)
            fire_writes(c, 1, g1)
            drain_writes(1)
            fire_gathers(1, g1 + 2)
            return carry

        lax.fori_loop(0, _NGC // 2 - 1, pair, 0)

        g0 = _NGC - 2
        drain_gathers(0)
        fire_writes(c, 0, g0)
        drain_gathers(1)
        drain_writes(0)
        fire_writes(c, 1, g0 + 1)
        drain_writes(1)


def kernel(indices, table):
    n = indices.shape[0]
    idx_pad = jnp.pad(indices.astype(jnp.int32), ((0, 0), (0, _TP - _T)))
    idx_flat = idx_pad.reshape(-1)
    mesh = plsc.VectorSubcoreMesh(core_axis_name="c", subcore_axis_name="s")
    f = functools.partial(
        pl.kernel,
        out_type=jax.ShapeDtypeStruct((n, _T, _D), jnp.float32),
        mesh=mesh,
        scratch_types=[
            pltpu.VMEM((_CHUNK * _TP,), jnp.int32),
            pltpu.VMEM((2 * _K * _TP, _D), jnp.float32),
            pltpu.SemaphoreType.DMA,
            pltpu.SemaphoreType.DMA,
            pltpu.SemaphoreType.DMA,
            pltpu.SemaphoreType.DMA,
        ],
    )(_sc_body)
    return f(idx_flat, table)


# speedup vs baseline: 8.0234x; 8.0234x over previous
import jax
import jax.numpy as jnp
from jax import lax
from jax.experimental import pallas as pl

_RB = 512
_T = 100
_STYLE_DIM = 128
_VPAD = 32


def _tc_body(idx_ref, tab_ref, out_ref):
    tab = tab_ref[...]
    for i in range(_RB):
        idx = idx_ref[i, :]
        iota = lax.broadcasted_iota(jnp.int32, (_VPAD, _T), 0)
        onehot = (iota == idx[None, :]).astype(jnp.float32)
        out_ref[i] = lax.dot_general(
            onehot, tab,
            (((0,), (0,)), ((), ())),
            preferred_element_type=jnp.float32,
        )


def kernel(indices, table):
    n = indices.shape[0]
    nblk = n // _RB
    idx = indices.astype(jnp.int32)
    tab = jnp.zeros((_VPAD, _STYLE_DIM), table.dtype).at[:table.shape[0]].set(table)
    return pl.pallas_call(
        _tc_body,
        grid=(nblk,),
        in_specs=[
            pl.BlockSpec((_RB, _T), lambda i: (i, 0)),
            pl.BlockSpec((_VPAD, _STYLE_DIM), lambda i: (0, 0)),
        ],
        out_specs=pl.BlockSpec((_RB, _T, _STYLE_DIM), lambda i: (i, 0, 0)),
        out_shape=jax.ShapeDtypeStruct((n, _T, _STYLE_DIM), jnp.float32),
    )(idx, tab)
